# baseline (speedup 1.0000x reference)
import functools

import jax
import jax.numpy as jnp
from jax.experimental import pallas as pl
from jax.experimental.pallas import tpu as pltpu

_TOP_P = 0.95
_NUM_SAMPLES = 5
_TEMPERATURE = 1.0
_B = 128
_V = 100000
_NPAD = 131072
_SUB = _NPAD // 128


def _sort_body(x_ref, o_ref):
    a = x_ref[0]
    r = jax.lax.broadcasted_iota(jnp.int32, (_SUB, 128), 0)
    c = jax.lax.broadcasted_iota(jnp.int32, (_SUB, 128), 1)
    for kb in range(1, 18):
        k = 1 << kb
        for jb in range(kb - 1, -1, -1):
            j = 1 << jb
            if j >= _SUB:
                s = j // _SUB
                lower = (c & s) == 0
                partner = jnp.where(lower, pltpu.roll(a, 128 - s, 1),
                                    pltpu.roll(a, s, 1))
            else:
                lower = (r & j) == 0
                partner = jnp.where(lower, pltpu.roll(a, _SUB - j, 0),
                                    pltpu.roll(a, j, 0))
            if k >= _NPAD:
                dir_up = jnp.zeros((_SUB, 128), dtype=jnp.bool_)
            elif k >= _SUB:
                dir_up = (c & (k // _SUB)) != 0
            else:
                dir_up = (r & k) != 0
            mn = jnp.minimum(a, partner)
            mx = jnp.maximum(a, partner)
            a = jnp.where(lower == dir_up, mn, mx)
    o_ref[0] = a


def _lane_cumsum(x, c):
    for sh in (1, 2, 4, 8, 16, 32, 64):
        x = x + jnp.where(c >= sh, pltpu.roll(x, sh, 1), 0)
    return x


def _sub_cumsum(x, r):
    for sh in (1, 2, 4, 8, 16, 32, 64, 128, 256, 512):
        x = x + jnp.where(r >= sh, pltpu.roll(x, sh, 0), 0)
    return x


def _sample_body(sv_ref, cs_ref, u_ref, p_ref, o_ref):
    sv = sv_ref[0]
    cs = cs_ref[0]
    uu = u_ref[0]
    pp = p_ref[0]
    r = jax.lax.broadcasted_iota(jnp.int32, (_SUB, 128), 0)
    c = jax.lax.broadcasted_iota(jnp.int32, (_SUB, 128), 1)
    jidx = c * _SUB + r
    vidx = r * 128 + c

    keep = ((cs - sv) <= _TOP_P) & (sv > 0.0)
    g = -jnp.log(-jnp.log(uu))
    fkey = -1e30 - jidx.astype(jnp.float32) * 1e25
    key = jnp.where(keep, jnp.log(jnp.where(keep, sv, 1.0)) + g, fkey)

    lane = jax.lax.broadcasted_iota(jnp.int32, (1, 128), 1)
    out_vec = jnp.zeros((1, 128), jnp.int32)
    bigi = jnp.int32(2147483647)
    for t in range(_NUM_SAMPLES):
        m = jnp.max(key)
        jstar = jnp.min(jnp.where(key == m, jidx, bigi))
        sel = jidx == jstar
        vstar = jnp.sum(jnp.where(sel, sv, 0.0))
        eqm = pp == vstar
        eq = eqm.astype(jnp.int32)
        neq = jnp.sum(eq)

        def _tie_break(eq=eq, vstar=vstar, jstar=jstar):
            cgt = jnp.sum(jnp.where(pp > vstar, 1, 0).astype(jnp.int32))
            trank = jstar - cgt
            ex_lane = _lane_cumsum(eq, c) - eq
            rowt = jnp.sum(eq, axis=1, keepdims=True)
            rowt_b = jnp.broadcast_to(rowt, (_SUB, 128))
            row_excl = _sub_cumsum(rowt_b, r) - rowt_b
            excl = ex_lane + row_excl
            return jnp.min(
                jnp.where((eq == 1) & (excl == trank), vidx, bigi))

        def _unique(eqm=eqm):
            return jnp.min(jnp.where(eqm, vidx, bigi))

        tok = jax.lax.cond(neq == 1, _unique, _tie_break)
        out_vec = jnp.where(lane == t, tok, out_vec)
        key = jnp.where(sel, jnp.float32(-3e30), key)
    o_ref[0] = out_vec


def _row_spec():
    return pl.BlockSpec((1, _SUB, 128), lambda i: (i, 0, 0))


def _to_cm(x, fill):
    xp = jnp.pad(x, ((0, 0), (0, _NPAD - _V)), constant_values=fill)
    return xp.reshape(_B, 128, _SUB).transpose(0, 2, 1)


@functools.partial(jax.jit, static_argnames=("interpret",))
def kernel(logits, u, interpret=False):
    probs = jax.nn.softmax(logits / _TEMPERATURE, axis=-1)
    p3 = jnp.pad(probs, ((0, 0), (0, _NPAD - _V)),
                 constant_values=-1.0).reshape(_B, _SUB, 128)

    sv3 = pl.pallas_call(
        _sort_body,
        grid=(_B,),
        in_specs=[_row_spec()],
        out_specs=_row_spec(),
        out_shape=jax.ShapeDtypeStruct((_B, _SUB, 128), jnp.float32),
        compiler_params=pltpu.CompilerParams(
            dimension_semantics=("parallel",)),
        interpret=interpret,
    )(p3)

    sv = sv3.transpose(0, 2, 1).reshape(_B, _NPAD)[:, :_V]
    cs = jnp.cumsum(sv, axis=-1)
    cs3 = _to_cm(cs, 3.0)
    u3 = _to_cm(u, 0.5)

    out = pl.pallas_call(
        _sample_body,
        grid=(_B,),
        in_specs=[_row_spec(), _row_spec(), _row_spec(), _row_spec()],
        out_specs=pl.BlockSpec((1, 1, 128), lambda i: (i, 0, 0)),
        out_shape=jax.ShapeDtypeStruct((_B, 1, 128), jnp.int32),
        compiler_params=pltpu.CompilerParams(
            dimension_semantics=("parallel",)),
        interpret=interpret,
    )(sv3, cs3, u3, p3)

    return out[:, 0, :_NUM_SAMPLES]


# speedup vs baseline: 1.0269x; 1.0269x over previous
import functools

import jax
import jax.numpy as jnp
from jax.experimental import pallas as pl
from jax.experimental.pallas import tpu as pltpu

_TOP_P = 0.95
_NUM_SAMPLES = 5
_TEMPERATURE = 1.0
_B = 128
_V = 100000
_NPAD = 131072
_SUB = _NPAD // 128


def _sort_body(x_ref, o_ref):
    a = x_ref[0]
    r = jax.lax.broadcasted_iota(jnp.int32, (_SUB, 128), 0)
    c = jax.lax.broadcasted_iota(jnp.int32, (_SUB, 128), 1)
    for kb in range(1, 18):
        k = 1 << kb
        for jb in range(kb - 1, -1, -1):
            j = 1 << jb
            if 8 <= j < _SUB:
                g = _SUB // (2 * j)
                v = a.reshape(g, 2, j, 128)
                lo = v[:, 0]
                hi = v[:, 1]
                if k >= _SUB:
                    dir_up = (jax.lax.broadcasted_iota(
                        jnp.int32, (g, j, 128), 2) & (k // _SUB)) != 0
                else:
                    dir_up = (jax.lax.broadcasted_iota(
                        jnp.int32, (g, j, 128), 0) & (k // (2 * j))) != 0
                mn = jnp.minimum(lo, hi)
                mx = jnp.maximum(lo, hi)
                res_lo = jnp.where(dir_up, mn, mx)
                res_hi = jnp.where(dir_up, mx, mn)
                a = jnp.concatenate(
                    [res_lo[:, None], res_hi[:, None]],
                    axis=1).reshape(_SUB, 128)
                continue
            if j >= _SUB:
                s = j // _SUB
                lower = (c & s) == 0
                partner = jnp.where(lower, pltpu.roll(a, 128 - s, 1),
                                    pltpu.roll(a, s, 1))
            else:
                lower = (r & j) == 0
                partner = jnp.where(lower, pltpu.roll(a, _SUB - j, 0),
                                    pltpu.roll(a, j, 0))
            if k >= _NPAD:
                dir_up = jnp.zeros((_SUB, 128), dtype=jnp.bool_)
            elif k >= _SUB:
                dir_up = (c & (k // _SUB)) != 0
            else:
                dir_up = (r & k) != 0
            mn = jnp.minimum(a, partner)
            mx = jnp.maximum(a, partner)
            a = jnp.where(lower == dir_up, mn, mx)
    o_ref[0] = a


def _lane_cumsum(x, c):
    for sh in (1, 2, 4, 8, 16, 32, 64):
        x = x + jnp.where(c >= sh, pltpu.roll(x, sh, 1), 0)
    return x


def _sub_cumsum(x, r):
    for sh in (1, 2, 4, 8, 16, 32, 64, 128, 256, 512):
        x = x + jnp.where(r >= sh, pltpu.roll(x, sh, 0), 0)
    return x


def _sample_body(sv_ref, cs_ref, u_ref, p_ref, o_ref):
    sv = sv_ref[0]
    cs = cs_ref[0]
    uu = u_ref[0]
    pp = p_ref[0]
    r = jax.lax.broadcasted_iota(jnp.int32, (_SUB, 128), 0)
    c = jax.lax.broadcasted_iota(jnp.int32, (_SUB, 128), 1)
    jidx = c * _SUB + r

    keep = ((cs - sv) <= _TOP_P) & (sv > 0.0)
    g = -jnp.log(-jnp.log(uu))
    fkey = -1e30 - jidx.astype(jnp.float32) * 1e25
    key = jnp.where(keep, jnp.log(jnp.where(keep, sv, 1.0)) + g, fkey)

    lane = jax.lax.broadcasted_iota(jnp.int32, (1, 128), 1)
    out_vec = jnp.zeros((1, 128), jnp.int32)
    bigi = jnp.int32(2147483647)
    for t in range(_NUM_SAMPLES):
        m = jnp.max(key)
        jstar = jnp.min(jnp.where(key == m, jidx, bigi))
        sel = jidx == jstar
        vstar = jnp.sum(jnp.where(sel, sv, 0.0))
        eqm = pp == vstar
        eq = eqm.astype(jnp.int32)
        neq = jnp.sum(eq)

        def _tie_break(eq=eq, vstar=vstar, jstar=jstar):
            cgt = jnp.sum(jnp.where(pp > vstar, 1, 0).astype(jnp.int32))
            trank = jstar - cgt
            ex_sub = _sub_cumsum(eq, r) - eq
            cols = jnp.sum(eq, axis=0, keepdims=True)
            cols_b = jnp.broadcast_to(cols, (_SUB, 128))
            col_excl = _lane_cumsum(cols_b, c) - cols_b
            excl = ex_sub + col_excl
            return jnp.min(
                jnp.where((eq == 1) & (excl == trank), jidx, bigi))

        def _unique(eqm=eqm):
            return jnp.min(jnp.where(eqm, jidx, bigi))

        tok = jax.lax.cond(neq == 1, _unique, _tie_break)
        out_vec = jnp.where(lane == t, tok, out_vec)
        key = jnp.where(sel, jnp.float32(-3e30), key)
    o_ref[0] = out_vec


def _row_spec():
    return pl.BlockSpec((1, _SUB, 128), lambda i: (i, 0, 0))


def _to_cm(x, fill):
    xp = jnp.pad(x, ((0, 0), (0, _NPAD - _V)), constant_values=fill)
    return xp.reshape(_B, 128, _SUB).transpose(0, 2, 1)


@functools.partial(jax.jit, static_argnames=("interpret",))
def kernel(logits, u, interpret=False):
    probs = jax.nn.softmax(logits / _TEMPERATURE, axis=-1)
    p3 = _to_cm(probs, -1.0)

    sv3 = pl.pallas_call(
        _sort_body,
        grid=(_B,),
        in_specs=[_row_spec()],
        out_specs=_row_spec(),
        out_shape=jax.ShapeDtypeStruct((_B, _SUB, 128), jnp.float32),
        compiler_params=pltpu.CompilerParams(
            dimension_semantics=("parallel",)),
        interpret=interpret,
    )(p3)

    sv = sv3.transpose(0, 2, 1).reshape(_B, _NPAD)[:, :_V]
    cs = jnp.cumsum(sv, axis=-1)
    cs3 = _to_cm(cs, 3.0)
    u3 = _to_cm(u, 0.5)

    out = pl.pallas_call(
        _sample_body,
        grid=(_B,),
        in_specs=[_row_spec(), _row_spec(), _row_spec(), _row_spec()],
        out_specs=pl.BlockSpec((1, 1, 128), lambda i: (i, 0, 0)),
        out_shape=jax.ShapeDtypeStruct((_B, 1, 128), jnp.int32),
        compiler_params=pltpu.CompilerParams(
            dimension_semantics=("parallel",)),
        interpret=interpret,
    )(sv3, cs3, u3, p3)

    return out[:, 0, :_NUM_SAMPLES]


# speedup vs baseline: 1.0590x; 1.0312x over previous
import functools

import jax
import jax.numpy as jnp
from jax.experimental import pallas as pl
from jax.experimental.pallas import tpu as pltpu

_TOP_P = 0.95
_NUM_SAMPLES = 5
_TEMPERATURE = 1.0
_B = 128
_V = 100000
_NPAD = 131072
_SUB = _NPAD // 128


def _sort_body(x_ref, o_ref):
    a = x_ref[0]
    r = jax.lax.broadcasted_iota(jnp.int32, (_SUB, 128), 0)
    c = jax.lax.broadcasted_iota(jnp.int32, (_SUB, 128), 1)
    for kb in range(1, 18):
        k = 1 << kb
        for jb in range(kb - 1, -1, -1):
            j = 1 << jb
            if 8 <= j < _SUB:
                g = _SUB // (2 * j)
                v = a.reshape(g, 2, j, 128)
                lo = v[:, 0]
                hi = v[:, 1]
                if k >= _SUB:
                    dir_up = (jax.lax.broadcasted_iota(
                        jnp.int32, (g, j, 128), 2) & (k // _SUB)) != 0
                else:
                    dir_up = (jax.lax.broadcasted_iota(
                        jnp.int32, (g, j, 128), 0) & (k // (2 * j))) != 0
                mn = jnp.minimum(lo, hi)
                mx = jnp.maximum(lo, hi)
                res_lo = jnp.where(dir_up, mn, mx)
                res_hi = jnp.where(dir_up, mx, mn)
                a = jnp.concatenate(
                    [res_lo[:, None], res_hi[:, None]],
                    axis=1).reshape(_SUB, 128)
                continue
            if j >= _SUB:
                s = j // _SUB
                lower = (c & s) == 0
                partner = jnp.where(lower, pltpu.roll(a, 128 - s, 1),
                                    pltpu.roll(a, s, 1))
            else:
                lower = (r & j) == 0
                partner = jnp.where(lower, pltpu.roll(a, _SUB - j, 0),
                                    pltpu.roll(a, j, 0))
            if k >= _NPAD:
                dir_up = jnp.zeros((_SUB, 128), dtype=jnp.bool_)
            elif k >= _SUB:
                dir_up = (c & (k // _SUB)) != 0
            else:
                dir_up = (r & k) != 0
            mn = jnp.minimum(a, partner)
            mx = jnp.maximum(a, partner)
            a = jnp.where(lower == dir_up, mn, mx)
    o_ref[0] = a


def _lane_cumsum(x, c):
    for sh in (1, 2, 4, 8, 16, 32, 64):
        x = x + jnp.where(c >= sh, pltpu.roll(x, sh, 1), 0)
    return x


def _sub_cumsum(x, r):
    for sh in (1, 2, 4, 8, 16, 32, 64, 128, 256, 512):
        x = x + jnp.where(r >= sh, pltpu.roll(x, sh, 0), 0)
    return x


def _sample_body(sv_ref, cs_ref, u_ref, p_ref, ji_ref, fk_ref, o_ref):
    sv = sv_ref[0]
    cs = cs_ref[0]
    uu = u_ref[0]
    pp = p_ref[0]
    jidx = ji_ref[0]
    fkey = fk_ref[0]

    keep = ((cs - sv) <= _TOP_P) & (sv > 0.0)
    g = -jnp.log(-jnp.log(uu))
    key = jnp.where(keep, jnp.log(jnp.where(keep, sv, 1.0)) + g, fkey)

    lane = jax.lax.broadcasted_iota(jnp.int32, (1, 128), 1)
    out_vec = jnp.zeros((1, 128), jnp.int32)
    bigi = jnp.int32(2147483647)
    for t in range(_NUM_SAMPLES):
        m = jnp.max(key)
        ksel = key == m
        jstar = jnp.min(jnp.where(ksel, jidx, bigi))
        vstar = jnp.max(jnp.where(ksel, sv, -1.0))
        eqm = pp == vstar
        eq = eqm.astype(jnp.int32)
        neq = jnp.sum(eq)

        def _tie_break(eq=eq, vstar=vstar, jstar=jstar):
            r = jax.lax.broadcasted_iota(jnp.int32, (_SUB, 128), 0)
            c = jax.lax.broadcasted_iota(jnp.int32, (_SUB, 128), 1)
            cgt = jnp.sum(jnp.where(pp > vstar, 1, 0).astype(jnp.int32))
            trank = jstar - cgt
            ex_sub = _sub_cumsum(eq, r) - eq
            cols = jnp.sum(eq, axis=0, keepdims=True)
            cols_b = jnp.broadcast_to(cols, (_SUB, 128))
            col_excl = _lane_cumsum(cols_b, c) - cols_b
            excl = ex_sub + col_excl
            return jnp.min(
                jnp.where((eq == 1) & (excl == trank), jidx, bigi))

        def _unique(eqm=eqm):
            return jnp.min(jnp.where(eqm, jidx, bigi))

        tok = jax.lax.cond(neq == 1, _unique, _tie_break)
        out_vec = jnp.where(lane == t, tok, out_vec)
        key = jnp.where(jidx == jstar, jnp.float32(-3e30), key)
    o_ref[0] = out_vec


def _row_spec():
    return pl.BlockSpec((1, _SUB, 128), lambda i: (i, 0, 0))


def _to_cm(x, fill):
    xp = jnp.pad(x, ((0, 0), (0, _NPAD - _V)), constant_values=fill)
    return xp.reshape(_B, 128, _SUB).transpose(0, 2, 1)


@functools.partial(jax.jit, static_argnames=("interpret",))
def kernel(logits, u, interpret=False):
    probs = jax.nn.softmax(logits / _TEMPERATURE, axis=-1)
    p3 = _to_cm(probs, -1.0)

    sv3 = pl.pallas_call(
        _sort_body,
        grid=(_B,),
        in_specs=[_row_spec()],
        out_specs=_row_spec(),
        out_shape=jax.ShapeDtypeStruct((_B, _SUB, 128), jnp.float32),
        compiler_params=pltpu.CompilerParams(
            dimension_semantics=("parallel",)),
        interpret=interpret,
    )(p3)

    sv = sv3.transpose(0, 2, 1).reshape(_B, _NPAD)[:, :_V]
    cs = jnp.cumsum(sv, axis=-1)
    cs3 = _to_cm(cs, 3.0)
    u3 = _to_cm(u, 0.5)

    rr = jax.lax.broadcasted_iota(jnp.int32, (1, _SUB, 128), 1)
    cc = jax.lax.broadcasted_iota(jnp.int32, (1, _SUB, 128), 2)
    ji = cc * _SUB + rr
    fk = -1e30 - ji.astype(jnp.float32) * 1e25

    _const_spec = pl.BlockSpec((1, _SUB, 128), lambda i: (0, 0, 0))
    out = pl.pallas_call(
        _sample_body,
        grid=(_B,),
        in_specs=[_row_spec(), _row_spec(), _row_spec(), _row_spec(),
                  _const_spec, _const_spec],
        out_specs=pl.BlockSpec((1, 1, 128), lambda i: (i, 0, 0)),
        out_shape=jax.ShapeDtypeStruct((_B, 1, 128), jnp.int32),
        compiler_params=pltpu.CompilerParams(
            dimension_semantics=("parallel",)),
        interpret=interpret,
    )(sv3, cs3, u3, p3, ji, fk)

    return out[:, 0, :_NUM_SAMPLES]
